# baseline (speedup 1.0000x reference)
import jax
import jax.numpy as jnp
from jax.experimental import pallas as pl
from jax.experimental.pallas import tpu as pltpu

VOCAB = 1024
HID = 1024
EMB = 256
NOS = 12
BN = 2048
HALF = 1024
SU = 1


def _rnn_body(target_ref, gum_ref, sos_ref, wih_ref, whh_ref, wout_ref,
              wemb_ref, bh_ref, bout_ref, bemb_ref, out_ref, *scr):
    nq = BN // HALF
    h_scrs, e_scrs = scr[:nq], scr[nq:]
    t = pl.program_id(1)
    is0 = t == 0

    def half_step(u, q):
        lo = q * HALF
        hi = lo + HALF
        h_scr, e_scr = h_scrs[q], e_scrs[q]
        if u == 0:
            e = jnp.where(is0,
                          jnp.broadcast_to(sos_ref[...], (HALF, EMB)),
                          e_scr[...])
            h_prev = jnp.where(is0, target_ref[lo:hi], h_scr[...])
        else:
            e = e_scr[...]
            h_prev = h_scr[...]
        pre = (jnp.dot(e, wih_ref[...], preferred_element_type=jnp.float32)
               + jnp.dot(h_prev, whh_ref[...],
                         preferred_element_type=jnp.float32)
               + bh_ref[...])
        h = jnp.tanh(pre).astype(jnp.bfloat16)
        logits = (jnp.dot(h, wout_ref[...], preferred_element_type=jnp.float32)
                  + bout_ref[...] + gum_ref[u, lo:hi])
        ex = jnp.exp(logits)
        h_scr[...] = h
        exb = ex.astype(jnp.bfloat16)
        rs = 1.0 / jnp.sum(ex, axis=-1, keepdims=True)
        e_scr[...] = (jnp.dot(exb, wemb_ref[...],
                              preferred_element_type=jnp.float32) * rs
                      + bemb_ref[...]).astype(jnp.bfloat16)
        out_ref[u, lo:hi] = ex * rs

    for u in range(SU):
        for q in range(BN // HALF):
            half_step(u, q)


def kernel(target, gumbels, sos, W_ih, b_ih, W_hh, b_hh, W_out, b_out,
           W_emb, b_emb):
    b_, m_, h_ = target.shape
    n = b_ * m_
    target2d = target.reshape(n, h_)
    grid = (n // BN, NOS // SU)

    out_flat = pl.pallas_call(
        _rnn_body,
        grid=grid,
        in_specs=[
            pl.BlockSpec((BN, HID), lambda i, t: (i, 0)),
            pl.BlockSpec((SU, BN, VOCAB), lambda i, t: (t, i, 0)),
            pl.BlockSpec((1, EMB), lambda i, t: (0, 0)),
            pl.BlockSpec((EMB, HID), lambda i, t: (0, 0)),
            pl.BlockSpec((HID, HID), lambda i, t: (0, 0)),
            pl.BlockSpec((HID, VOCAB), lambda i, t: (0, 0)),
            pl.BlockSpec((VOCAB, EMB), lambda i, t: (0, 0)),
            pl.BlockSpec((1, HID), lambda i, t: (0, 0)),
            pl.BlockSpec((1, VOCAB), lambda i, t: (0, 0)),
            pl.BlockSpec((1, EMB), lambda i, t: (0, 0)),
        ],
        out_specs=pl.BlockSpec((SU, BN, VOCAB), lambda i, t: (t, i, 0)),
        out_shape=jax.ShapeDtypeStruct((NOS, n, VOCAB), jnp.float32),
        scratch_shapes=(
            [pltpu.VMEM((HALF, HID), jnp.bfloat16)
             for _ in range(BN // HALF)]
            + [pltpu.VMEM((HALF, EMB), jnp.bfloat16)
               for _ in range(BN // HALF)]
        ),
        compiler_params=pltpu.CompilerParams(
            dimension_semantics=("parallel", "arbitrary"),
            vmem_limit_bytes=56 * 1024 * 1024,
            flags={"XLA_TPU_STORE_TO_LOAD_FORWARDING_WINDOW": 12288},
        ),
    )(target2d.astype(jnp.bfloat16), gumbels,
      sos.reshape(1, EMB).astype(jnp.bfloat16),
      W_ih.T.astype(jnp.bfloat16), W_hh.T.astype(jnp.bfloat16),
      W_out.T.astype(jnp.bfloat16), W_emb.T.astype(jnp.bfloat16),
      (b_ih + b_hh).reshape(1, HID), b_out.reshape(1, VOCAB),
      b_emb.reshape(1, EMB))

    return jnp.transpose(out_flat, (1, 0, 2))


# speedup vs baseline: 1.0028x; 1.0028x over previous
import jax
import jax.numpy as jnp
from jax.experimental import pallas as pl
from jax.experimental.pallas import tpu as pltpu

VOCAB = 1024
HID = 1024
EMB = 256
NOS = 12
BN = 2048
HALF = 1024
SU = 1


def _rnn_body(target_ref, gum_ref, sos_ref, wih_ref, whh_ref, wout_ref,
              wemb_ref, bh_ref, bout_ref, bemb_ref, out_ref, *scr):
    nq = BN // HALF
    h_scrs, e_scrs = scr[:nq], scr[nq:]
    t = pl.program_id(1)
    is0 = t == 0

    def half_step(u, q):
        lo = q * HALF
        hi = lo + HALF
        h_scr, e_scr = h_scrs[q], e_scrs[q]
        if u == 0:
            e = jnp.where(is0,
                          jnp.broadcast_to(sos_ref[...], (HALF, EMB)),
                          e_scr[...])
            h_prev = jnp.where(is0, target_ref[lo:hi], h_scr[...])
        else:
            e = e_scr[...]
            h_prev = h_scr[...]
        pre = (jnp.dot(e, wih_ref[...], preferred_element_type=jnp.float32)
               + jnp.dot(h_prev, whh_ref[...],
                         preferred_element_type=jnp.float32)
               + bh_ref[...])
        h = jnp.tanh(pre).astype(jnp.bfloat16)
        logits = (jnp.dot(h, wout_ref[...], preferred_element_type=jnp.float32)
                  + bout_ref[...] + gum_ref[u, lo:hi])
        ex = jnp.exp(logits)
        h_scr[...] = h
        exb = ex.astype(jnp.bfloat16)
        rs = 1.0 / jnp.sum(ex, axis=-1, keepdims=True)
        e_scr[...] = (jnp.dot(exb, wemb_ref[...],
                              preferred_element_type=jnp.float32) * rs
                      + bemb_ref[...]).astype(jnp.bfloat16)
        out_ref[u, lo:hi] = ex * rs

    for u in range(SU):
        for q in range(BN // HALF):
            half_step(u, q)


def kernel(target, gumbels, sos, W_ih, b_ih, W_hh, b_hh, W_out, b_out,
           W_emb, b_emb):
    b_, m_, h_ = target.shape
    n = b_ * m_
    target2d = target.reshape(n, h_)
    grid = (n // BN, NOS // SU)

    out_flat = pl.pallas_call(
        _rnn_body,
        grid=grid,
        in_specs=[
            pl.BlockSpec((BN, HID), lambda i, t: (i, 0)),
            pl.BlockSpec((SU, BN, VOCAB), lambda i, t: (t, i, 0)),
            pl.BlockSpec((1, EMB), lambda i, t: (0, 0)),
            pl.BlockSpec((EMB, HID), lambda i, t: (0, 0)),
            pl.BlockSpec((HID, HID), lambda i, t: (0, 0)),
            pl.BlockSpec((HID, VOCAB), lambda i, t: (0, 0)),
            pl.BlockSpec((VOCAB, EMB), lambda i, t: (0, 0)),
            pl.BlockSpec((1, HID), lambda i, t: (0, 0)),
            pl.BlockSpec((1, VOCAB), lambda i, t: (0, 0)),
            pl.BlockSpec((1, EMB), lambda i, t: (0, 0)),
        ],
        out_specs=pl.BlockSpec((SU, BN, VOCAB), lambda i, t: (t, i, 0)),
        out_shape=jax.ShapeDtypeStruct((NOS, n, VOCAB), jnp.float32),
        scratch_shapes=(
            [pltpu.VMEM((HALF, HID), jnp.bfloat16)
             for _ in range(BN // HALF)]
            + [pltpu.VMEM((HALF, EMB), jnp.bfloat16)
               for _ in range(BN // HALF)]
        ),
        compiler_params=pltpu.CompilerParams(
            dimension_semantics=("parallel", "arbitrary"),
            vmem_limit_bytes=56 * 1024 * 1024,
        ),
    )(target2d.astype(jnp.bfloat16), gumbels,
      sos.reshape(1, EMB).astype(jnp.bfloat16),
      W_ih.T.astype(jnp.bfloat16), W_hh.T.astype(jnp.bfloat16),
      W_out.T.astype(jnp.bfloat16), W_emb.T.astype(jnp.bfloat16),
      (b_ih + b_hh).reshape(1, HID), b_out.reshape(1, VOCAB),
      b_emb.reshape(1, EMB))

    return jnp.transpose(out_flat, (1, 0, 2))
